# baseline (speedup 1.0000x reference)
import functools

import jax
import jax.numpy as jnp
from jax import lax
from jax.experimental import pallas as pl
from jax.experimental.pallas import tpu as pltpu
from jax.experimental.pallas import tpu_sc as plsc

HIDDEN = 32
NC, NS = 2, 16
NW = NC * NS
NBUF = 4


def _emb_kernel(n_rows, n_cols):
    rows_w = n_rows // NW
    mesh = plsc.VectorSubcoreMesh(
        core_axis_name="c", subcore_axis_name="s",
        num_cores=NC, num_subcores=NS)

    @functools.partial(
        pl.kernel,
        out_type=jax.ShapeDtypeStruct((n_rows, n_cols, HIDDEN), jnp.float32),
        mesh=mesh,
        scratch_types=[
            pltpu.VMEM((rows_w, n_cols), jnp.int32),
            pltpu.VMEM((NBUF, n_cols, HIDDEN), jnp.float32),
        ] + [pltpu.SemaphoreType.DMA] * (2 * NBUF),
        compiler_params=pltpu.CompilerParams(use_tc_tiling_on_sc=False,
                                             needs_layout_passes=False),
    )
    def body(w_hbm, x_hbm, out_hbm, idx_v, rows_v, *sems):
        gsem = sems[:NBUF]
        wsem = sems[NBUF:]
        wid = lax.axis_index("s") * NC + lax.axis_index("c")
        base = wid * rows_w

        pltpu.sync_copy(x_hbm.at[pl.ds(base, rows_w)], idx_v)

        def g_start(b, r):
            pltpu.async_copy(w_hbm.at[idx_v.at[r]], rows_v.at[b], gsem[b])

        def g_wait(b, r):
            pltpu.make_async_copy(
                w_hbm.at[idx_v.at[r]], rows_v.at[b], gsem[b]).wait()

        def w_start(b, r):
            pltpu.async_copy(rows_v.at[b], out_hbm.at[base + r], wsem[b])

        def w_wait(b, r):
            pltpu.make_async_copy(
                rows_v.at[b], out_hbm.at[base + r], wsem[b]).wait()

        for b in range(NBUF):
            g_start(b, b)

        @pl.loop(0, rows_w, step=NBUF)
        def _(r):
            for b in range(NBUF):
                g_wait(b, r + b)
                w_start(b, r + b)
            for b in range(NBUF):
                nxt = r + b + NBUF

                @pl.when(nxt < rows_w)
                def _():
                    w_wait(b, r + b)
                    g_start(b, nxt)

        for b in range(NBUF):
            w_wait(b, rows_w - NBUF + b)

    return body


def kernel(x, weight):
    s0, s1 = x.shape
    return _emb_kernel(s0, s1)(weight, x.astype(jnp.int32))


# speedup vs baseline: 1.0444x; 1.0444x over previous
import functools

import jax
import jax.numpy as jnp
from jax import lax
from jax.experimental import pallas as pl
from jax.experimental.pallas import tpu as pltpu
from jax.experimental.pallas import tpu_sc as plsc

HIDDEN = 32
NC, NS = 2, 16
NW = NC * NS
NBUF = 4
GROUP = 2


def _emb_kernel(n_rows, n_cols):
    rows_w = n_rows // NW
    units_w = rows_w // GROUP
    gcols = GROUP * n_cols
    mesh = plsc.VectorSubcoreMesh(
        core_axis_name="c", subcore_axis_name="s",
        num_cores=NC, num_subcores=NS)

    @functools.partial(
        pl.kernel,
        out_type=jax.ShapeDtypeStruct((n_rows, n_cols, HIDDEN), jnp.float32),
        mesh=mesh,
        scratch_types=[
            pltpu.VMEM((units_w, gcols), jnp.int32),
            pltpu.VMEM((NBUF, gcols, HIDDEN), jnp.float32),
        ] + [pltpu.SemaphoreType.DMA] * (2 * NBUF),
        compiler_params=pltpu.CompilerParams(use_tc_tiling_on_sc=False),
    )
    def body(w_hbm, xg_hbm, out_hbm, idx_v, rows_v, *sems):
        gsem = sems[:NBUF]
        wsem = sems[NBUF:]
        wid = lax.axis_index("s") * NC + lax.axis_index("c")
        base = wid * rows_w

        pltpu.sync_copy(xg_hbm.at[pl.ds(wid * units_w, units_w)], idx_v)

        def g_start(b, u):
            pltpu.async_copy(w_hbm.at[idx_v.at[u]], rows_v.at[b], gsem[b])

        def g_wait(b, u):
            pltpu.make_async_copy(
                w_hbm.at[idx_v.at[u]], rows_v.at[b], gsem[b]).wait()

        def w_start(b, u):
            for g in range(GROUP):
                pltpu.async_copy(
                    rows_v.at[b, pl.ds(g * n_cols, n_cols)],
                    out_hbm.at[base + u * GROUP + g], wsem[b])

        def w_wait(b, u):
            for g in range(GROUP):
                pltpu.make_async_copy(
                    rows_v.at[b, pl.ds(g * n_cols, n_cols)],
                    out_hbm.at[base + u * GROUP + g], wsem[b]).wait()

        for b in range(NBUF):
            g_start(b, b)

        @pl.loop(0, units_w, step=NBUF)
        def _(u):
            for b in range(NBUF):
                g_wait(b, u + b)
                w_start(b, u + b)
            for b in range(NBUF):
                nxt = u + b + NBUF

                @pl.when(nxt < units_w)
                def _():
                    w_wait(b, u + b)
                    g_start(b, nxt)

        for b in range(NBUF):
            w_wait(b, units_w - NBUF + b)

    return body


def kernel(x, weight):
    s0, s1 = x.shape
    xg = x.astype(jnp.int32).reshape(s0 // GROUP, GROUP * s1)
    return _emb_kernel(s0, s1)(weight, xg)
